# baseline (speedup 1.0000x reference)
import jax
import jax.numpy as jnp
from jax.experimental import pallas as pl
from jax.experimental.pallas import tpu as pltpu

_CROP = 24
_CB = 8


def _axis(lo, hi, size, crop):
    scale = (hi - lo) * (size - 1.0) / (crop - 1.0)
    c = lo[:, None] * (size - 1.0) + (
        jnp.arange(crop, dtype=jnp.float32)[None, :] * scale[:, None]
    )
    valid = (c >= 0.0) & (c <= size - 1.0)
    c0 = jnp.floor(c)
    frac = c - c0
    i0 = jnp.clip(c0.astype(jnp.int32), 0, size - 1)
    i1 = jnp.clip(i0 + 1, 0, size - 1)
    return i0, i1, frac, valid


def _weight_matrix(i0, i1, frac, valid, size):
    oh0 = jax.nn.one_hot(i0, size, dtype=jnp.float32)
    oh1 = jax.nn.one_hot(i1, size, dtype=jnp.float32)
    w = oh0 * (1.0 - frac)[..., None] + oh1 * frac[..., None]
    return w * valid.astype(jnp.float32)[..., None]


def _interp_kernel(ord_ref, bis_ref, z0_ref, wz_ref, wy_ref, wxt_ref,
                   v0_ref, v1_ref, v2_ref, v3_ref, out_ref):
    b = pl.program_id(1)
    wyb = wy_ref[0]
    wxtb = wxt_ref[0]
    vrefs = (v0_ref, v1_ref, v2_ref, v3_ref)
    for z in range(_CROP):
        zi = z0_ref[b, z]
        p0 = jnp.concatenate([v[0, 0, :, zi, :, :] for v in vrefs], axis=0)
        p1 = jnp.concatenate([v[0, 0, :, zi + 1, :, :] for v in vrefs], axis=0)
        p = (p0 * wz_ref[b, z, 0] + p1 * wz_ref[b, z, 1]).astype(jnp.bfloat16)
        s = jax.lax.dot_general(p.reshape(_CB * 64, 64), wxtb,
                                (((1,), (0,)), ((), ())),
                                preferred_element_type=jnp.float32)
        s3 = s.astype(jnp.bfloat16).reshape(_CB, 64, _CROP)
        for ci in range(_CB):
            qv = jax.lax.dot_general(wyb, s3[ci], (((1,), (0,)), ((), ())),
                                     preferred_element_type=jnp.float32)
            out_ref[0, ci, z, :, :] = qv


def kernel(image, boxes, box_ind):
    n, c, d, h, w = image.shape
    bz1, by1, bx1, bz2, by2, bx2 = (boxes[:, i] for i in range(6))
    z0, z1, fz, vz = _axis(bz1, bz2, d, _CROP)
    y0, y1, fy, vy = _axis(by1, by2, h, _CROP)
    x0, x1, fx, vx = _axis(bx1, bx2, w, _CROP)

    z0s = jnp.minimum(z0, d - 2)
    fzs = fz + (z0 - z0s).astype(jnp.float32)
    vzf = vz.astype(jnp.float32)
    wz = jnp.stack([(1.0 - fzs) * vzf, fzs * vzf], axis=-1)
    wy = _weight_matrix(y0, y1, fy, vy, h).astype(jnp.bfloat16)
    wxt = jnp.transpose(_weight_matrix(x0, x1, fx, vx, w),
                        (0, 2, 1)).astype(jnp.bfloat16)

    bi = box_ind.astype(jnp.int32)
    nb = boxes.shape[0]
    order = jnp.argsort(bi).astype(jnp.int32)

    img6 = image.reshape(n, c // 2, 2, d, h, w)

    def vol_spec(g):
        def imap(cb, b, ord_, bis_, z0_, wz_):
            return (bis_[b], cb * 4 + g, 0, 0, 0, 0)
        return pl.BlockSpec((1, 1, 2, d, h, w), imap)

    grid_spec = pltpu.PrefetchScalarGridSpec(
        num_scalar_prefetch=4,
        grid=(c // _CB, nb),
        in_specs=[
            pl.BlockSpec((1, _CROP, h), lambda cb, b, *sp: (b, 0, 0)),
            pl.BlockSpec((1, w, _CROP), lambda cb, b, *sp: (b, 0, 0)),
        ] + [vol_spec(g) for g in range(4)],
        out_specs=pl.BlockSpec((1, _CB, _CROP, _CROP, _CROP),
                               lambda cb, b, ord_, bis_, z0_, wz_: (ord_[b], cb, 0, 0, 0)),
    )

    return pl.pallas_call(
        _interp_kernel,
        grid_spec=grid_spec,
        out_shape=jax.ShapeDtypeStruct((nb, c, _CROP, _CROP, _CROP), jnp.float32),
        compiler_params=pltpu.CompilerParams(
            dimension_semantics=("arbitrary", "arbitrary"),
        ),
    )(order, bi[order], z0s[order], wz[order], wy[order], wxt[order],
      img6, img6, img6, img6)


# speedup vs baseline: 1.0061x; 1.0061x over previous
import jax
import jax.numpy as jnp
from jax.experimental import pallas as pl
from jax.experimental.pallas import tpu as pltpu

_CROP = 24
_CB = 8


def _axis(lo, hi, size, crop):
    scale = (hi - lo) * (size - 1.0) / (crop - 1.0)
    c = lo[:, None] * (size - 1.0) + (
        jnp.arange(crop, dtype=jnp.float32)[None, :] * scale[:, None]
    )
    valid = (c >= 0.0) & (c <= size - 1.0)
    c0 = jnp.floor(c)
    frac = c - c0
    i0 = jnp.clip(c0.astype(jnp.int32), 0, size - 1)
    i1 = jnp.clip(i0 + 1, 0, size - 1)
    return i0, i1, frac, valid


def _weight_matrix(i0, i1, frac, valid, size):
    oh0 = jax.nn.one_hot(i0, size, dtype=jnp.float32)
    oh1 = jax.nn.one_hot(i1, size, dtype=jnp.float32)
    w = oh0 * (1.0 - frac)[..., None] + oh1 * frac[..., None]
    return w * valid.astype(jnp.float32)[..., None]


def _interp_kernel(ord_ref, bis_ref, z0_ref, wz_ref, wy_ref, wxt_ref,
                   vol_ref, out_ref):
    b = pl.program_id(1)
    wyb = wy_ref[0]
    wxtb = wxt_ref[0]
    for z in range(_CROP):
        zi = z0_ref[b, z]
        p0 = vol_ref[0, :, zi, :, :]
        p1 = vol_ref[0, :, zi + 1, :, :]
        p = (p0 * wz_ref[b, z, 0] + p1 * wz_ref[b, z, 1]).astype(jnp.bfloat16)
        s = jax.lax.dot_general(p.reshape(_CB * 64, 64), wxtb,
                                (((1,), (0,)), ((), ())),
                                preferred_element_type=jnp.float32)
        s3 = s.astype(jnp.bfloat16).reshape(_CB, 64, _CROP)
        for ci in range(_CB):
            qv = jax.lax.dot_general(wyb, s3[ci], (((1,), (0,)), ((), ())),
                                     preferred_element_type=jnp.float32)
            out_ref[0, ci, z, :, :] = qv


def kernel(image, boxes, box_ind):
    n, c, d, h, w = image.shape
    bz1, by1, bx1, bz2, by2, bx2 = (boxes[:, i] for i in range(6))
    z0, z1, fz, vz = _axis(bz1, bz2, d, _CROP)
    y0, y1, fy, vy = _axis(by1, by2, h, _CROP)
    x0, x1, fx, vx = _axis(bx1, bx2, w, _CROP)

    z0s = jnp.minimum(z0, d - 2)
    fzs = fz + (z0 - z0s).astype(jnp.float32)
    vzf = vz.astype(jnp.float32)
    wz = jnp.stack([(1.0 - fzs) * vzf, fzs * vzf], axis=-1)
    wy = _weight_matrix(y0, y1, fy, vy, h).astype(jnp.bfloat16)
    wxt = jnp.transpose(_weight_matrix(x0, x1, fx, vx, w),
                        (0, 2, 1)).astype(jnp.bfloat16)

    bi = box_ind.astype(jnp.int32)
    nb = boxes.shape[0]
    order = jnp.argsort(bi).astype(jnp.int32)

    grid_spec = pltpu.PrefetchScalarGridSpec(
        num_scalar_prefetch=4,
        grid=(c // _CB, nb),
        in_specs=[
            pl.BlockSpec((1, _CROP, h), lambda cb, b, *sp: (b, 0, 0)),
            pl.BlockSpec((1, w, _CROP), lambda cb, b, *sp: (b, 0, 0)),
            pl.BlockSpec((1, _CB, d, h, w),
                         lambda cb, b, ord_, bis_, z0_, wz_: (bis_[b], cb, 0, 0, 0)),
        ],
        out_specs=pl.BlockSpec((1, _CB, _CROP, _CROP, _CROP),
                               lambda cb, b, ord_, bis_, z0_, wz_: (ord_[b], cb, 0, 0, 0)),
    )

    return pl.pallas_call(
        _interp_kernel,
        grid_spec=grid_spec,
        out_shape=jax.ShapeDtypeStruct((nb, c, _CROP, _CROP, _CROP), jnp.float32),
        compiler_params=pltpu.CompilerParams(
            dimension_semantics=("arbitrary", "arbitrary"),
        ),
    )(order, bi[order], z0s[order], wz[order], wy[order], wxt[order], image)
